# Initial kernel scaffold; baseline (speedup 1.0000x reference)
import jax
import jax.numpy as jnp
from jax.experimental import pallas as pl


def kernel(features, edge_index, edge_weight, W0_0, W0_1, W0_2, W1_0, W1_1, W1_2, W2_0, W2_1, W2_2, Wc1, bc1, bn_g, bn_b, Wc2, bc2):
    raise NotImplementedError("write your pallas kernel here")


# speedup vs baseline: 6.5124x; 6.5124x over previous
import functools

import jax
import jax.numpy as jnp
import numpy as np
from jax import lax
from jax.experimental import pallas as pl
from jax.experimental.pallas import tpu as pltpu
from jax.experimental.pallas import tpu_sc as plsc

_N = 10000
_E = 320000
_D = 128
_CH = 200
_NCHUNK = _E // _CH
_TILES = 32
_CPT = _NCHUNK // _TILES
_SLAB = _N // 16
_NP = 10240
_DSLAB = _NP // 16
_GSTARTS = list(range(0, _CH - 15, 16)) + ([_CH - 16] if _CH % 16 else [])
_NGRP_FULL = _CH // 16
_TAIL = _CH - 16 * _NGRP_FULL


def _mesh():
    return plsc.VectorSubcoreMesh(core_axis_name="c", subcore_axis_name="s")


def _rsqrt16(d):
    pos = d > 0.0
    dc = jnp.where(pos, d, 1.0)
    i = lax.bitcast_convert_type(dc, jnp.int32)
    i = jnp.int32(0x5F3759DF) - lax.shift_right_arithmetic(i, 1)
    y = lax.bitcast_convert_type(i, jnp.float32)
    for _ in range(3):
        y = y * (1.5 - 0.5 * dc * y * y)
    return jnp.where(pos, y, 0.0)


def _deg_body(src2, dst2, w2, zeros, deg_out, acc, srcb, dstb, wb, wzb):
    c = lax.axis_index("c")
    s = lax.axis_index("s")
    t = c * 16 + s
    pltpu.sync_copy(zeros, acc.at[pl.ds(s * _DSLAB, _DSLAB)])
    plsc.subcore_barrier()

    def chunk(i, carry):
        cid = t * _CPT + i
        pltpu.sync_copy(src2.at[cid], srcb)
        pltpu.sync_copy(dst2.at[cid], dstb)
        pltpu.sync_copy(w2.at[cid], wb)
        for gs in _GSTARTS:
            sl = pl.ds(gs, 16)
            wzb[sl] = jnp.where(srcb[sl] == dstb[sl], 0.0, wb[sl])
        pltpu.sync_copy(wzb, acc.at[srcb], add=True)
        return carry

    lax.fori_loop(0, _CPT, chunk, 0)
    plsc.subcore_barrier()
    pltpu.sync_copy(acc.at[pl.ds(s * _DSLAB, _DSLAB)], deg_out.at[c, s])


def _dinv_body(deg, dinv_out, d0b, d1b, obuf):
    c = lax.axis_index("c")
    s = lax.axis_index("s")

    @pl.when(c == 0)
    def _():
        pltpu.sync_copy(deg.at[0, s], d0b)
        pltpu.sync_copy(deg.at[1, s], d1b)
        for k in range(_DSLAB // 16):
            sl = pl.ds(k * 16, 16)
            obuf[sl] = _rsqrt16(d0b[sl] + d1b[sl])
        pltpu.sync_copy(obuf, dinv_out.at[s])


def _norm_body(src2, dst2, w2, dinv2, norm_out, dinvb, srcb, dstb, wb, nb):
    c = lax.axis_index("c")
    s = lax.axis_index("s")
    t = c * 16 + s
    for k in range(16):
        pltpu.sync_copy(dinv2.at[k], dinvb.at[pl.ds(k * _DSLAB, _DSLAB)])

    def chunk(i, carry):
        cid = t * _CPT + i
        pltpu.sync_copy(src2.at[cid], srcb)
        pltpu.sync_copy(dst2.at[cid], dstb)
        pltpu.sync_copy(w2.at[cid], wb)
        for gs in _GSTARTS:
            sl = pl.ds(gs, 16)
            sv = srcb[sl]
            dv = dstb[sl]
            wv = wb[sl]
            wz = jnp.where(sv == dv, 0.0, wv)
            ds_ = plsc.load_gather(dinvb, [sv])
            dd_ = plsc.load_gather(dinvb, [dv])
            nb[sl] = -(ds_ * wz * dd_)
        pltpu.sync_copy(nb, norm_out.at[cid])
        return carry

    lax.fori_loop(0, _CPT, chunk, 0)


def _spmm_body(x, src2, dst2, norm2, zeros, p_out, acc, rows, srcb, dstb,
               normb, sem):
    c = lax.axis_index("c")
    s = lax.axis_index("s")
    t = c * 16 + s
    pltpu.sync_copy(zeros, acc.at[pl.ds(s * _SLAB, _SLAB)])
    plsc.subcore_barrier()

    def chunk(i, carry):
        cid = t * _CPT + i
        pltpu.sync_copy(src2.at[cid], srcb)
        pltpu.sync_copy(dst2.at[cid], dstb)
        pltpu.sync_copy(norm2.at[cid], normb)
        pltpu.async_copy(x.at[srcb], rows, sem).wait()

        def grp(g, carry2):
            nvv = normb[pl.ds(g * 16, 16)]
            for k in range(16):
                nv = nvv[k]
                r = g * 16 + k
                for j in range(_D // 16):
                    sl = pl.ds(j * 16, 16)
                    rows[r, sl] = rows[r, sl] * nv
            return carry2

        lax.fori_loop(0, _NGRP_FULL, grp, 0)
        if _TAIL:
            nvt = normb[pl.ds(_CH - 16, 16)]
            for k in range(_TAIL):
                nv = nvt[16 - _TAIL + k]
                r = 16 * _NGRP_FULL + k
                for j in range(_D // 16):
                    sl = pl.ds(j * 16, 16)
                    rows[r, sl] = rows[r, sl] * nv
        pltpu.sync_copy(rows, acc.at[dstb], add=True)
        return carry

    lax.fori_loop(0, _CPT, chunk, 0)
    plsc.subcore_barrier()
    pltpu.sync_copy(acc.at[pl.ds(s * _SLAB, _SLAB)], p_out.at[c, s])


def _run_deg(src2, dst2, w2, zeros):
    f = pl.kernel(
        _deg_body,
        out_type=jax.ShapeDtypeStruct((2, 16, _DSLAB), jnp.float32),
        mesh=_mesh(),
        compiler_params=pltpu.CompilerParams(needs_layout_passes=False),
        scratch_types=[
            pltpu.VMEM_SHARED((_NP,), jnp.float32),
            pltpu.VMEM((_CH,), jnp.int32),
            pltpu.VMEM((_CH,), jnp.int32),
            pltpu.VMEM((_CH,), jnp.float32),
            pltpu.VMEM((_CH,), jnp.float32),
        ],
    )
    return f(src2, dst2, w2, zeros)


def _run_dinv(deg):
    f = pl.kernel(
        _dinv_body,
        out_type=jax.ShapeDtypeStruct((16, _DSLAB), jnp.float32),
        mesh=_mesh(),
        compiler_params=pltpu.CompilerParams(needs_layout_passes=False),
        scratch_types=[
            pltpu.VMEM((_DSLAB,), jnp.float32),
            pltpu.VMEM((_DSLAB,), jnp.float32),
            pltpu.VMEM((_DSLAB,), jnp.float32),
        ],
    )
    return f(deg)


def _run_norm(src2, dst2, w2, dinv2):
    f = pl.kernel(
        _norm_body,
        out_type=jax.ShapeDtypeStruct((_NCHUNK, _CH), jnp.float32),
        mesh=_mesh(),
        compiler_params=pltpu.CompilerParams(needs_layout_passes=False),
        scratch_types=[
            pltpu.VMEM((_NP,), jnp.float32),
            pltpu.VMEM((_CH,), jnp.int32),
            pltpu.VMEM((_CH,), jnp.int32),
            pltpu.VMEM((_CH,), jnp.float32),
            pltpu.VMEM((_CH,), jnp.float32),
        ],
    )
    return f(src2, dst2, w2, dinv2)


def _run_spmm(x, src2, dst2, norm2, zeros):
    f = pl.kernel(
        _spmm_body,
        out_type=jax.ShapeDtypeStruct((2, 16, _SLAB, _D), jnp.float32),
        mesh=_mesh(),
        compiler_params=pltpu.CompilerParams(needs_layout_passes=False),
        scratch_types=[
            pltpu.VMEM_SHARED((_N, _D), jnp.float32),
            pltpu.VMEM((_CH, _D), jnp.float32),
            pltpu.VMEM((_CH,), jnp.int32),
            pltpu.VMEM((_CH,), jnp.int32),
            pltpu.VMEM((_CH,), jnp.float32),
            pltpu.SemaphoreType.DMA,
        ],
    )
    return f(x, src2, dst2, norm2, zeros)


_BLK = 1000


def _tc1_kern(x_ref, p0_ref, p1_ref, w0_ref, w1_ref, tx1_ref, acc_ref):
    tx1 = p0_ref[...] + p1_ref[...]
    tx1_ref[...] = tx1
    acc_ref[...] = (
        jnp.dot(x_ref[...], w0_ref[...], preferred_element_type=jnp.float32)
        + jnp.dot(tx1, w1_ref[...], preferred_element_type=jnp.float32))


def _tc1(x, p0, p1, w0, w1):
    grid = (_N // _BLK,)
    row = pl.BlockSpec((_BLK, _D), lambda i: (i, 0))
    full = pl.BlockSpec((_D, _D), lambda i: (0, 0))
    return pl.pallas_call(
        _tc1_kern,
        grid=grid,
        in_specs=[row, row, row, full, full],
        out_specs=[row, row],
        out_shape=[jax.ShapeDtypeStruct((_N, _D), jnp.float32),
                   jax.ShapeDtypeStruct((_N, _D), jnp.float32)],
    )(x, p0, p1, w0, w1)


def _tc2_kern(acc_ref, x_ref, q0_ref, q1_ref, w2_ref, out_ref):
    tx2 = 2.0 * (q0_ref[...] + q1_ref[...]) - x_ref[...]
    out_ref[...] = jnp.maximum(
        acc_ref[...]
        + jnp.dot(tx2, w2_ref[...], preferred_element_type=jnp.float32), 0.0)


def _tc2(acc, x, q0, q1, w2):
    grid = (_N // _BLK,)
    row = pl.BlockSpec((_BLK, _D), lambda i: (i, 0))
    full = pl.BlockSpec((_D, _D), lambda i: (0, 0))
    return pl.pallas_call(
        _tc2_kern,
        grid=grid,
        in_specs=[row, row, row, row, full],
        out_specs=row,
        out_shape=jax.ShapeDtypeStruct((_N, _D), jnp.float32),
    )(acc, x, q0, q1, w2)


_BN_INV = float(1.0 / np.sqrt(1.0 + 1e-5))


def _head_kern(x_ref, wc1_ref, bc1_ref, g_ref, b_ref, wc2_ref, bc2_ref,
               out_ref):
    h = jnp.maximum(
        jnp.dot(x_ref[...], wc1_ref[...], preferred_element_type=jnp.float32)
        + bc1_ref[...], 0.0)
    h = h * (g_ref[...] * _BN_INV) + b_ref[...]
    out_ref[...] = (
        jnp.dot(h, wc2_ref[...], preferred_element_type=jnp.float32)
        + bc2_ref[...])


def _head(x, wc1, bc1, bn_g, bn_b, wc2p, bc2p):
    grid = (_N // _BLK,)
    row = pl.BlockSpec((_BLK, _D), lambda i: (i, 0))
    return pl.pallas_call(
        _head_kern,
        grid=grid,
        in_specs=[
            row,
            pl.BlockSpec((_D, 256), lambda i: (0, 0)),
            pl.BlockSpec((1, 256), lambda i: (0, 0)),
            pl.BlockSpec((1, 256), lambda i: (0, 0)),
            pl.BlockSpec((1, 256), lambda i: (0, 0)),
            pl.BlockSpec((256, _D), lambda i: (0, 0)),
            pl.BlockSpec((1, _D), lambda i: (0, 0)),
        ],
        out_specs=pl.BlockSpec((_BLK, _D), lambda i: (i, 0)),
        out_shape=jax.ShapeDtypeStruct((_N, _D), jnp.float32),
    )(x, wc1, bc1, bn_g, bn_b, wc2p, bc2p)


def kernel(features, edge_index, edge_weight, W0_0, W0_1, W0_2, W1_0, W1_1,
           W1_2, W2_0, W2_1, W2_2, Wc1, bc1, bn_g, bn_b, Wc2, bc2):
    src2 = edge_index[0].reshape(_NCHUNK, _CH)
    dst2 = edge_index[1].reshape(_NCHUNK, _CH)
    w2 = edge_weight.reshape(_NCHUNK, _CH)
    zeros = jnp.zeros((_DSLAB,), jnp.float32)
    zeros_r = jnp.zeros((_SLAB, _D), jnp.float32)

    deg = _run_deg(src2, dst2, w2, zeros)
    dinv2 = _run_dinv(deg)
    norm2 = _run_norm(src2, dst2, w2, dinv2)

    x = features
    for (w0, w1, w2_) in ((W0_0, W0_1, W0_2), (W1_0, W1_1, W1_2),
                          (W2_0, W2_1, W2_2)):
        p = _run_spmm(x, src2, dst2, norm2, zeros_r)
        tx1, acc = _tc1(x, p[0].reshape(_N, _D), p[1].reshape(_N, _D), w0, w1)
        q = _run_spmm(tx1, src2, dst2, norm2, zeros_r)
        x = _tc2(acc, x, q[0].reshape(_N, _D), q[1].reshape(_N, _D), w2_)

    wc2p = jnp.zeros((256, _D), jnp.float32).at[:, :Wc2.shape[1]].set(Wc2)
    bc2p = jnp.zeros((1, _D), jnp.float32).at[0, :bc2.shape[0]].set(bc2)
    logit_pad = _head(x, Wc1, bc1.reshape(1, 256), bn_g.reshape(1, 256),
                      bn_b.reshape(1, 256), wc2p, bc2p)
    return (logit_pad[:, :Wc2.shape[1]], edge_weight)


# speedup vs baseline: 7.9724x; 1.2242x over previous
import functools

import jax
import jax.numpy as jnp
import numpy as np
from jax import lax
from jax.experimental import pallas as pl
from jax.experimental.pallas import tpu as pltpu
from jax.experimental.pallas import tpu_sc as plsc

_N = 10000
_E = 320000
_D = 128
_CH = 100
_NCHUNK = _E // _CH
_TILES = 32
_CPT = _NCHUNK // _TILES
_SLAB = _N // 16
_NP = 10240
_DSLAB = _NP // 16
_GSTARTS = list(range(0, _CH - 15, 16)) + ([_CH - 16] if _CH % 16 else [])
_NGRP_FULL = _CH // 16
_TAIL = _CH - 16 * _NGRP_FULL


def _mesh():
    return plsc.VectorSubcoreMesh(core_axis_name="c", subcore_axis_name="s")


def _rsqrt16(d):
    pos = d > 0.0
    dc = jnp.where(pos, d, 1.0)
    i = lax.bitcast_convert_type(dc, jnp.int32)
    i = jnp.int32(0x5F3759DF) - lax.shift_right_arithmetic(i, 1)
    y = lax.bitcast_convert_type(i, jnp.float32)
    for _ in range(3):
        y = y * (1.5 - 0.5 * dc * y * y)
    return jnp.where(pos, y, 0.0)


def _deg_body(src2, dst2, w2, zeros, deg_out, acc, srcb, dstb, wb, wzb):
    c = lax.axis_index("c")
    s = lax.axis_index("s")
    t = c * 16 + s
    pltpu.sync_copy(zeros, acc.at[pl.ds(s * _DSLAB, _DSLAB)])
    plsc.subcore_barrier()

    def chunk(i, carry):
        cid = t * _CPT + i
        pltpu.sync_copy(src2.at[cid], srcb)
        pltpu.sync_copy(dst2.at[cid], dstb)
        pltpu.sync_copy(w2.at[cid], wb)
        for gs in _GSTARTS:
            sl = pl.ds(gs, 16)
            wzb[sl] = jnp.where(srcb[sl] == dstb[sl], 0.0, wb[sl])
        pltpu.sync_copy(wzb, acc.at[srcb], add=True)
        return carry

    lax.fori_loop(0, _CPT, chunk, 0)
    plsc.subcore_barrier()
    pltpu.sync_copy(acc.at[pl.ds(s * _DSLAB, _DSLAB)], deg_out.at[c, s])


def _dinv_body(deg, dinv_out, d0b, d1b, obuf):
    c = lax.axis_index("c")
    s = lax.axis_index("s")

    @pl.when(c == 0)
    def _():
        pltpu.sync_copy(deg.at[0, s], d0b)
        pltpu.sync_copy(deg.at[1, s], d1b)
        for k in range(_DSLAB // 16):
            sl = pl.ds(k * 16, 16)
            obuf[sl] = _rsqrt16(d0b[sl] + d1b[sl])
        pltpu.sync_copy(obuf, dinv_out.at[s])


def _norm_body(src2, dst2, w2, dinv2, norm_out, dinvb, srcb, dstb, wb, nb):
    c = lax.axis_index("c")
    s = lax.axis_index("s")
    t = c * 16 + s
    for k in range(16):
        pltpu.sync_copy(dinv2.at[k], dinvb.at[pl.ds(k * _DSLAB, _DSLAB)])

    def chunk(i, carry):
        cid = t * _CPT + i
        pltpu.sync_copy(src2.at[cid], srcb)
        pltpu.sync_copy(dst2.at[cid], dstb)
        pltpu.sync_copy(w2.at[cid], wb)
        for gs in _GSTARTS:
            sl = pl.ds(gs, 16)
            sv = srcb[sl]
            dv = dstb[sl]
            wv = wb[sl]
            wz = jnp.where(sv == dv, 0.0, wv)
            ds_ = plsc.load_gather(dinvb, [sv])
            dd_ = plsc.load_gather(dinvb, [dv])
            nb[sl] = -(ds_ * wz * dd_)
        pltpu.sync_copy(nb, norm_out.at[cid])
        return carry

    lax.fori_loop(0, _CPT, chunk, 0)


def _scale_rows(rows, normb):
    def grp(g, carry2):
        nvv = normb[pl.ds(g * 16, 16)]
        for k in range(16):
            nv = nvv[k]
            r = g * 16 + k
            for j in range(_D // 16):
                sl = pl.ds(j * 16, 16)
                rows[r, sl] = rows[r, sl] * nv
        return carry2

    lax.fori_loop(0, _NGRP_FULL, grp, 0)
    if _TAIL:
        nvt = normb[pl.ds(_CH - 16, 16)]
        for k in range(_TAIL):
            nv = nvt[16 - _TAIL + k]
            r = 16 * _NGRP_FULL + k
            for j in range(_D // 16):
                sl = pl.ds(j * 16, 16)
                rows[r, sl] = rows[r, sl] * nv


def _spmm_body(x, src2, dst2, norm2, zeros, p_out, acc, rows_a, rows_b,
               srcb_a, dstb_a, normb_a, srcb_b, dstb_b, normb_b,
               gsem_a, gsem_b, isem_a, isem_b, ssem_a, ssem_b):
    c = lax.axis_index("c")
    s = lax.axis_index("s")
    t = c * 16 + s
    c0 = t * _CPT
    slot_a = (rows_a, srcb_a, dstb_a, normb_a, gsem_a, isem_a, ssem_a)
    slot_b = (rows_b, srcb_b, dstb_b, normb_b, gsem_b, isem_b, ssem_b)

    pltpu.sync_copy(src2.at[c0], srcb_a)
    pltpu.sync_copy(dst2.at[c0], dstb_a)
    pltpu.sync_copy(norm2.at[c0], normb_a)
    pltpu.async_copy(x.at[srcb_a], rows_a, gsem_a)
    pltpu.sync_copy(zeros, acc.at[pl.ds(s * _SLAB, _SLAB)])
    plsc.subcore_barrier()

    def do(i, slot, slot_o, wait_prev, has_next):
        rows, srcb, dstb, normb, gsem, isem, ssem = slot
        rows_o, srcb_o, dstb_o, normb_o, gsem_o, isem_o, ssem_o = slot_o

        def _maybe(pred, fn):
            if pred is True:
                fn()
            else:
                pl.when(pred)(fn)

        _maybe(wait_prev, lambda: pltpu.make_async_copy(
            rows_o, acc.at[dstb_o], ssem_o).wait())

        def prefetch_idx():
            pltpu.async_copy(src2.at[i + 1], srcb_o, isem_o)
            pltpu.async_copy(dst2.at[i + 1], dstb_o, isem_o)
            pltpu.async_copy(norm2.at[i + 1], normb_o, isem_o)

        _maybe(has_next, prefetch_idx)
        pltpu.make_async_copy(x.at[srcb], rows, gsem).wait()
        _scale_rows(rows, normb)
        pltpu.async_copy(rows, acc.at[dstb], ssem, add=True)

        def next_gather():
            pltpu.make_async_copy(src2.at[i + 1], srcb_o, isem_o).wait()
            pltpu.make_async_copy(dst2.at[i + 1], dstb_o, isem_o).wait()
            pltpu.make_async_copy(norm2.at[i + 1], normb_o, isem_o).wait()
            pltpu.async_copy(x.at[srcb_o], rows_o, gsem_o)

        _maybe(has_next, next_gather)

    def pair(p, carry):
        i = c0 + 2 * p
        do(i, slot_a, slot_b, wait_prev=(p > 0), has_next=True)
        do(i + 1, slot_b, slot_a, wait_prev=True,
           has_next=(p < _CPT // 2 - 1))
        return carry

    lax.fori_loop(0, _CPT // 2, pair, 0)
    pltpu.make_async_copy(rows_b, acc.at[dstb_b], ssem_b).wait()
    plsc.subcore_barrier()
    pltpu.sync_copy(acc.at[pl.ds(s * _SLAB, _SLAB)], p_out.at[c, s])


def _run_deg(src2, dst2, w2, zeros):
    f = pl.kernel(
        _deg_body,
        out_type=jax.ShapeDtypeStruct((2, 16, _DSLAB), jnp.float32),
        mesh=_mesh(),
        compiler_params=pltpu.CompilerParams(needs_layout_passes=False),
        scratch_types=[
            pltpu.VMEM_SHARED((_NP,), jnp.float32),
            pltpu.VMEM((_CH,), jnp.int32),
            pltpu.VMEM((_CH,), jnp.int32),
            pltpu.VMEM((_CH,), jnp.float32),
            pltpu.VMEM((_CH,), jnp.float32),
        ],
    )
    return f(src2, dst2, w2, zeros)


def _run_dinv(deg):
    f = pl.kernel(
        _dinv_body,
        out_type=jax.ShapeDtypeStruct((16, _DSLAB), jnp.float32),
        mesh=_mesh(),
        compiler_params=pltpu.CompilerParams(needs_layout_passes=False),
        scratch_types=[
            pltpu.VMEM((_DSLAB,), jnp.float32),
            pltpu.VMEM((_DSLAB,), jnp.float32),
            pltpu.VMEM((_DSLAB,), jnp.float32),
        ],
    )
    return f(deg)


def _run_norm(src2, dst2, w2, dinv2):
    f = pl.kernel(
        _norm_body,
        out_type=jax.ShapeDtypeStruct((_NCHUNK, _CH), jnp.float32),
        mesh=_mesh(),
        compiler_params=pltpu.CompilerParams(needs_layout_passes=False),
        scratch_types=[
            pltpu.VMEM((_NP,), jnp.float32),
            pltpu.VMEM((_CH,), jnp.int32),
            pltpu.VMEM((_CH,), jnp.int32),
            pltpu.VMEM((_CH,), jnp.float32),
            pltpu.VMEM((_CH,), jnp.float32),
        ],
    )
    return f(src2, dst2, w2, dinv2)


def _run_spmm(x, src2, dst2, norm2, zeros):
    f = pl.kernel(
        _spmm_body,
        out_type=jax.ShapeDtypeStruct((2, 16, _SLAB, _D), jnp.float32),
        mesh=_mesh(),
        compiler_params=pltpu.CompilerParams(needs_layout_passes=False),
        scratch_types=[
            pltpu.VMEM_SHARED((_N, _D), jnp.float32),
            pltpu.VMEM((_CH, _D), jnp.float32),
            pltpu.VMEM((_CH, _D), jnp.float32),
            pltpu.VMEM((_CH,), jnp.int32),
            pltpu.VMEM((_CH,), jnp.int32),
            pltpu.VMEM((_CH,), jnp.float32),
            pltpu.VMEM((_CH,), jnp.int32),
            pltpu.VMEM((_CH,), jnp.int32),
            pltpu.VMEM((_CH,), jnp.float32),
            pltpu.SemaphoreType.DMA,
            pltpu.SemaphoreType.DMA,
            pltpu.SemaphoreType.DMA,
            pltpu.SemaphoreType.DMA,
            pltpu.SemaphoreType.DMA,
            pltpu.SemaphoreType.DMA,
        ],
    )
    return f(x, src2, dst2, norm2, zeros)


_BLK = 1000


def _tc1_kern(x_ref, p0_ref, p1_ref, w0_ref, w1_ref, tx1_ref, acc_ref):
    tx1 = p0_ref[...] + p1_ref[...]
    tx1_ref[...] = tx1
    acc_ref[...] = (
        jnp.dot(x_ref[...], w0_ref[...], preferred_element_type=jnp.float32)
        + jnp.dot(tx1, w1_ref[...], preferred_element_type=jnp.float32))


def _tc1(x, p0, p1, w0, w1):
    grid = (_N // _BLK,)
    row = pl.BlockSpec((_BLK, _D), lambda i: (i, 0))
    full = pl.BlockSpec((_D, _D), lambda i: (0, 0))
    return pl.pallas_call(
        _tc1_kern,
        grid=grid,
        in_specs=[row, row, row, full, full],
        out_specs=[row, row],
        out_shape=[jax.ShapeDtypeStruct((_N, _D), jnp.float32),
                   jax.ShapeDtypeStruct((_N, _D), jnp.float32)],
    )(x, p0, p1, w0, w1)


def _tc2_kern(acc_ref, x_ref, q0_ref, q1_ref, w2_ref, out_ref):
    tx2 = 2.0 * (q0_ref[...] + q1_ref[...]) - x_ref[...]
    out_ref[...] = jnp.maximum(
        acc_ref[...]
        + jnp.dot(tx2, w2_ref[...], preferred_element_type=jnp.float32), 0.0)


def _tc2(acc, x, q0, q1, w2):
    grid = (_N // _BLK,)
    row = pl.BlockSpec((_BLK, _D), lambda i: (i, 0))
    full = pl.BlockSpec((_D, _D), lambda i: (0, 0))
    return pl.pallas_call(
        _tc2_kern,
        grid=grid,
        in_specs=[row, row, row, row, full],
        out_specs=row,
        out_shape=jax.ShapeDtypeStruct((_N, _D), jnp.float32),
    )(acc, x, q0, q1, w2)


_BN_INV = float(1.0 / np.sqrt(1.0 + 1e-5))


def _head_kern(x_ref, wc1_ref, bc1_ref, g_ref, b_ref, wc2_ref, bc2_ref,
               out_ref):
    h = jnp.maximum(
        jnp.dot(x_ref[...], wc1_ref[...], preferred_element_type=jnp.float32)
        + bc1_ref[...], 0.0)
    h = h * (g_ref[...] * _BN_INV) + b_ref[...]
    out_ref[...] = (
        jnp.dot(h, wc2_ref[...], preferred_element_type=jnp.float32)
        + bc2_ref[...])


def _head(x, wc1, bc1, bn_g, bn_b, wc2p, bc2p):
    grid = (_N // _BLK,)
    row = pl.BlockSpec((_BLK, _D), lambda i: (i, 0))
    return pl.pallas_call(
        _head_kern,
        grid=grid,
        in_specs=[
            row,
            pl.BlockSpec((_D, 256), lambda i: (0, 0)),
            pl.BlockSpec((1, 256), lambda i: (0, 0)),
            pl.BlockSpec((1, 256), lambda i: (0, 0)),
            pl.BlockSpec((1, 256), lambda i: (0, 0)),
            pl.BlockSpec((256, _D), lambda i: (0, 0)),
            pl.BlockSpec((1, _D), lambda i: (0, 0)),
        ],
        out_specs=pl.BlockSpec((_BLK, _D), lambda i: (i, 0)),
        out_shape=jax.ShapeDtypeStruct((_N, _D), jnp.float32),
    )(x, wc1, bc1, bn_g, bn_b, wc2p, bc2p)


def kernel(features, edge_index, edge_weight, W0_0, W0_1, W0_2, W1_0, W1_1,
           W1_2, W2_0, W2_1, W2_2, Wc1, bc1, bn_g, bn_b, Wc2, bc2):
    src2 = edge_index[0].reshape(_NCHUNK, _CH)
    dst2 = edge_index[1].reshape(_NCHUNK, _CH)
    w2 = edge_weight.reshape(_NCHUNK, _CH)
    zeros = jnp.zeros((_DSLAB,), jnp.float32)
    zeros_r = jnp.zeros((_SLAB, _D), jnp.float32)

    deg = _run_deg(src2, dst2, w2, zeros)
    dinv2 = _run_dinv(deg)
    norm2 = _run_norm(src2, dst2, w2, dinv2)

    x = features
    for (w0, w1, w2_) in ((W0_0, W0_1, W0_2), (W1_0, W1_1, W1_2),
                          (W2_0, W2_1, W2_2)):
        p = _run_spmm(x, src2, dst2, norm2, zeros_r)
        tx1, acc = _tc1(x, p[0].reshape(_N, _D), p[1].reshape(_N, _D), w0, w1)
        q = _run_spmm(tx1, src2, dst2, norm2, zeros_r)
        x = _tc2(acc, x, q[0].reshape(_N, _D), q[1].reshape(_N, _D), w2_)

    wc2p = jnp.zeros((256, _D), jnp.float32).at[:, :Wc2.shape[1]].set(Wc2)
    bc2p = jnp.zeros((1, _D), jnp.float32).at[0, :bc2.shape[0]].set(bc2)
    logit_pad = _head(x, Wc1, bc1.reshape(1, 256), bn_g.reshape(1, 256),
                      bn_b.reshape(1, 256), wc2p, bc2p)
    return (logit_pad[:, :Wc2.shape[1]], edge_weight)
